# baseline (speedup 1.0000x reference)
import functools

import jax
import jax.numpy as jnp
from jax import lax
from jax.experimental import pallas as pl
from jax.experimental.pallas import tpu as pltpu
from jax.experimental.pallas import tpu_sc as plsc

_B, _H, _S, _D = 8, 16, 2048, 128
_Q = 32
_BH = _B * _H
_R = _BH * _Q
_NC, _NS = 2, 16
_NW = _NC * _NS
_RPW = _R // _NW
_BHPW = _BH // _NW


@functools.partial(
    pl.kernel,
    out_type=(),
    mesh=plsc.VectorSubcoreMesh(core_axis_name="c", subcore_axis_name="s"),
    scratch_types=[
        pltpu.VMEM((_Q,), jnp.int32),
        pltpu.VMEM((_RPW,), jnp.int32),
        pltpu.VMEM((_RPW, _D), jnp.float32),
        pltpu.SemaphoreType.DMA,
    ],
)
def _scatter_rows(pos_hbm, val_hbm, out_ref, pos_v, idx_v, rows_v, sem):
    wid = lax.axis_index("s") * _NC + lax.axis_index("c")
    base = wid * _RPW
    rcopy = pltpu.async_copy(val_hbm.at[pl.ds(base, _RPW)], rows_v, sem)
    pltpu.sync_copy(pos_hbm, pos_v)
    for i in range(_RPW // 16):
        q0 = (i % (_Q // 16)) * 16
        bh = wid * _BHPW + i // (_Q // 16)
        idx_v[pl.ds(i * 16, 16)] = pos_v[pl.ds(q0, 16)] + bh * _S
    rcopy.wait()
    pltpu.async_copy(rows_v, out_ref.at[idx_v], sem).wait()


def kernel(input_pos, k_val, v_val, k_cache, v_cache):
    pos = input_pos.astype(jnp.int32)
    kv = k_val.reshape(_R, _D)
    vv = v_val.reshape(_R, _D)
    ko = jax.new_ref(k_cache.reshape(_BH * _S, _D))
    _scatter_rows(pos, kv, ko)
    vo = jax.new_ref(v_cache.reshape(_BH * _S, _D))
    _scatter_rows(pos, vv, vo)
    return (ko[...].reshape(_B, _H, _S, _D),
            vo[...].reshape(_B, _H, _S, _D))


# speedup vs baseline: 1.0279x; 1.0279x over previous
import functools

import jax
import jax.numpy as jnp
from jax import lax
from jax.experimental import pallas as pl
from jax.experimental.pallas import tpu as pltpu
from jax.experimental.pallas import tpu_sc as plsc

_B, _H, _S, _D = 8, 16, 2048, 128
_Q = 32
_BH = _B * _H
_R = _BH * _Q
_NC, _NS = 2, 16
_NW = _NC * _NS
_RPW = _R // _NW
_BHPW = _BH // _NW


@functools.partial(
    pl.kernel,
    out_type=(),
    mesh=plsc.VectorSubcoreMesh(core_axis_name="c", subcore_axis_name="s"),
    scratch_types=[
        pltpu.VMEM((_Q,), jnp.int32),
        pltpu.VMEM((_RPW,), jnp.int32),
        pltpu.VMEM((_RPW, _D), jnp.float32),
        pltpu.VMEM((_RPW, _D), jnp.float32),
        pltpu.SemaphoreType.DMA,
        pltpu.SemaphoreType.DMA,
    ],
)
def _scatter_rows(pos_hbm, kv_hbm, vv_hbm, ko_ref, vo_ref,
                  pos_v, idx_v, krows_v, vrows_v, ksem, vsem):
    wid = lax.axis_index("s") * _NC + lax.axis_index("c")
    base = wid * _RPW
    kcopy = pltpu.async_copy(kv_hbm.at[pl.ds(base, _RPW)], krows_v, ksem)
    vcopy = pltpu.async_copy(vv_hbm.at[pl.ds(base, _RPW)], vrows_v, vsem)
    pltpu.sync_copy(pos_hbm, pos_v)
    for i in range(_RPW // 16):
        q0 = (i % (_Q // 16)) * 16
        bh = wid * _BHPW + i // (_Q // 16)
        idx_v[pl.ds(i * 16, 16)] = pos_v[pl.ds(q0, 16)] + bh * _S
    kcopy.wait()
    kscat = pltpu.async_copy(krows_v, ko_ref.at[idx_v], ksem)
    vcopy.wait()
    vscat = pltpu.async_copy(vrows_v, vo_ref.at[idx_v], vsem)
    kscat.wait()
    vscat.wait()


def kernel(input_pos, k_val, v_val, k_cache, v_cache):
    pos = input_pos.astype(jnp.int32)
    kv = k_val.reshape(_R, _D)
    vv = v_val.reshape(_R, _D)
    ko = jax.new_ref(k_cache.reshape(_BH * _S, _D))
    vo = jax.new_ref(v_cache.reshape(_BH * _S, _D))
    _scatter_rows(pos, kv, vv, ko, vo)
    return (ko[...].reshape(_B, _H, _S, _D),
            vo[...].reshape(_B, _H, _S, _D))
